# Initial kernel scaffold; baseline (speedup 1.0000x reference)
import jax
import jax.numpy as jnp
from jax.experimental import pallas as pl


def kernel(x, edge_index, W1, a_src1, a_dst1, b1, W2, a_src2, a_dst2, b2):
    raise NotImplementedError("write your pallas kernel here")


# speedup vs baseline: 43.0304x; 43.0304x over previous
import functools

import jax
import jax.numpy as jnp
from jax import lax
from jax.experimental import pallas as pl
from jax.experimental.pallas import tpu as pltpu
from jax.experimental.pallas import tpu_sc as plsc

L = 16


def _prep1_body(x_ref, w1_ref, g_ref, g2_ref, t1_ref, t2_ref):
  h = jnp.dot(x_ref[...], w1_ref[...], preferred_element_type=jnp.float32,
              precision=lax.Precision.HIGHEST)
  t1_ref[...] = jnp.dot(h, g_ref[...], preferred_element_type=jnp.float32,
                        precision=lax.Precision.HIGHEST)
  t2_ref[...] = jnp.dot(h, g2_ref[...], preferred_element_type=jnp.float32,
                        precision=lax.Precision.HIGHEST)


def _mid_body(p0_ref, p1_ref, s1_ref, s2_ref, b1_ref, w2e_ref, w2d_ref,
              u1e_ref, td2_ref):
  acc = p0_ref[...] + p1_ref[...]
  denex = jnp.dot(acc, s1_ref[...], preferred_element_type=jnp.float32,
                  precision=lax.Precision.HIGHEST)
  num = jnp.dot(acc, s2_ref[...], preferred_element_type=jnp.float32,
                precision=lax.Precision.HIGHEST)
  out1 = num / (denex + 1e-16) + b1_ref[...]
  hm = jnp.where(out1 > 0, out1, jnp.exp(out1) - 1.0)
  u1e_ref[...] = jnp.dot(hm, w2e_ref[...], preferred_element_type=jnp.float32,
                         precision=lax.Precision.HIGHEST)
  td2_ref[...] = jnp.dot(hm, w2d_ref[...], preferred_element_type=jnp.float32,
                         precision=lax.Precision.HIGHEST)


def _final_body(q0_ref, q1_ref, s3_ref, s4_ref, b2_ref, out_ref):
  acc = q0_ref[...] + q1_ref[...]
  num = jnp.dot(acc, s3_ref[...], preferred_element_type=jnp.float32,
                precision=lax.Precision.HIGHEST)
  den = jnp.dot(acc, s4_ref[...], preferred_element_type=jnp.float32,
                precision=lax.Precision.HIGHEST)
  o = num / (den + 1e-16) + b2_ref[...]
  m = jnp.max(o, axis=1, keepdims=True)
  ex = jnp.exp(o - m)
  out_ref[...] = ex / jnp.sum(ex, axis=1, keepdims=True)


def _lane_gather(v, idx):
  return jnp.take_along_axis(v, idx, axis=0, mode="promise_in_bounds")


def _edge1_kernel(n_pad, ept, k_chunk, t1, t2, ei_s, ei_d, zrows):
  n_ch = ept // k_chunk
  rows_per_tile = n_pad // 16
  mesh = plsc.VectorSubcoreMesh(core_axis_name="c", subcore_axis_name="s")

  @functools.partial(
      pl.kernel,
      out_type=jax.ShapeDtypeStruct((2, n_pad, 80), jnp.float32),
      mesh=mesh,
      compiler_params=pltpu.CompilerParams(use_tc_tiling_on_sc=False),
      scratch_types=[
          pltpu.VMEM((k_chunk,), jnp.int32),
          pltpu.VMEM((k_chunk,), jnp.int32),
          pltpu.VMEM((k_chunk, 80), jnp.float32),
          pltpu.VMEM((k_chunk, 16), jnp.float32),
          pltpu.VMEM((k_chunk, 80), jnp.float32),
          pltpu.VMEM_SHARED((n_pad, 80), jnp.float32),
      ],
  )
  def body(t1_hbm, t2_hbm, eis_hbm, eid_hbm, z_hbm, out_hbm,
           idx_s, idx_d, rows1, rowsd, stage, acc):
    c = lax.axis_index("c")
    s = lax.axis_index("s")
    wid = c * 16 + s
    r0 = s * rows_per_tile
    pltpu.sync_copy(z_hbm.at[pl.ds(0, rows_per_tile)],
                    acc.at[pl.ds(r0, rows_per_tile)])
    plsc.subcore_barrier()

    lane = lax.iota(jnp.int32, L)
    half = lane >> 3

    def edge_body(k, _):
      va = rows1[k, pl.ds(0, 16)]
      vd = rowsd[k]
      e = va + vd
      e = jnp.where(e < 0, 0.2 * e, e)
      w = jnp.exp(e)
      stage[k, pl.ds(0, 16)] = w
      for j in range(4):
        wj = _lane_gather(w, half + 2 * j)
        hj = rows1[k, pl.ds(16 + 16 * j, 16)]
        stage[k, pl.ds(16 + 16 * j, 16)] = wj * hj
      return 0

    def chunk_body(i, _):
      base = wid * ept + i * k_chunk
      pltpu.sync_copy(eis_hbm.at[pl.ds(base, k_chunk)], idx_s)
      pltpu.sync_copy(eid_hbm.at[pl.ds(base, k_chunk)], idx_d)
      pltpu.sync_copy(t1_hbm.at[idx_s], rows1)
      pltpu.sync_copy(t2_hbm.at[idx_d], rowsd)
      lax.fori_loop(0, k_chunk, edge_body, 0)
      pltpu.sync_copy(stage, acc.at[idx_d], add=True)
      return 0

    lax.fori_loop(0, n_ch, chunk_body, 0)
    plsc.subcore_barrier()
    pltpu.sync_copy(acc.at[pl.ds(r0, rows_per_tile)],
                    out_hbm.at[c, pl.ds(r0, rows_per_tile)])

  return body(t1, t2, ei_s, ei_d, zrows)


def _edge2_kernel(n_pad, ept, k_chunk, u1e, td2, ei_s, ei_d, zrows):
  n_ch = ept // k_chunk
  rows_per_tile = n_pad // 16
  mesh = plsc.VectorSubcoreMesh(core_axis_name="c", subcore_axis_name="s")

  @functools.partial(
      pl.kernel,
      out_type=jax.ShapeDtypeStruct((2, n_pad, 32), jnp.float32),
      mesh=mesh,
      compiler_params=pltpu.CompilerParams(use_tc_tiling_on_sc=False),
      scratch_types=[
          pltpu.VMEM((k_chunk,), jnp.int32),
          pltpu.VMEM((k_chunk,), jnp.int32),
          pltpu.VMEM((k_chunk, 32), jnp.float32),
          pltpu.VMEM((k_chunk, 16), jnp.float32),
          pltpu.VMEM((k_chunk, 32), jnp.float32),
          pltpu.VMEM_SHARED((n_pad, 32), jnp.float32),
      ],
  )
  def body(u1_hbm, td2_hbm, eis_hbm, eid_hbm, z_hbm, out_hbm,
           idx_s, idx_d, rowsu, rowsd2, stage, acc):
    c = lax.axis_index("c")
    s = lax.axis_index("s")
    wid = c * 16 + s
    r0 = s * rows_per_tile
    pltpu.sync_copy(z_hbm.at[pl.ds(0, rows_per_tile)],
                    acc.at[pl.ds(r0, rows_per_tile)])
    plsc.subcore_barrier()

    lane = lax.iota(jnp.int32, L)
    zero16 = lane >> 4

    def edge_body(k, _):
      vu = rowsu[k, pl.ds(0, 16)]
      va = rowsu[k, pl.ds(16, 16)]
      vd = rowsd2[k]
      e = va + vd
      e = jnp.where(e < 0, 0.2 * e, e)
      w = jnp.exp(e)
      wk = _lane_gather(w, zero16)
      stage[k, pl.ds(0, 16)] = wk * vu
      stage[k, pl.ds(16, 16)] = wk
      return 0

    def chunk_body(i, _):
      base = wid * ept + i * k_chunk
      pltpu.sync_copy(eis_hbm.at[pl.ds(base, k_chunk)], idx_s)
      pltpu.sync_copy(eid_hbm.at[pl.ds(base, k_chunk)], idx_d)
      pltpu.sync_copy(u1_hbm.at[idx_s], rowsu)
      pltpu.sync_copy(td2_hbm.at[idx_d], rowsd2)
      lax.fori_loop(0, k_chunk, edge_body, 0)
      pltpu.sync_copy(stage, acc.at[idx_d], add=True)
      return 0

    lax.fori_loop(0, n_ch, chunk_body, 0)
    plsc.subcore_barrier()
    pltpu.sync_copy(acc.at[pl.ds(r0, rows_per_tile)],
                    out_hbm.at[c, pl.ds(r0, rows_per_tile)])

  return body(u1e, td2, ei_s, ei_d, zrows)


def kernel(x, edge_index, W1, a_src1, a_dst1, b1, W2, a_src2, a_dst2, b2):
  n, d = x.shape
  e_cnt = edge_index.shape[1]
  hid = W1.shape[1]
  ncls = W2.shape[1]

  n_pad = ((n + 511) // 512) * 512
  ept = e_cnt // 32
  k_chunk = 80
  blk = 256
  n_blk = n_pad // blk

  ar = jnp.arange(hid)
  As1m = jnp.zeros((hid, 8), jnp.float32).at[ar, ar // 8].set(a_src1.reshape(-1))
  Ad1m = jnp.zeros((hid, 8), jnp.float32).at[ar, ar // 8].set(a_dst1.reshape(-1))
  G = jnp.concatenate([As1m, Ad1m, jnp.eye(hid, dtype=jnp.float32)], axis=1)
  G2 = jnp.concatenate([Ad1m, Ad1m], axis=1)
  S1 = jnp.zeros((80, hid), jnp.float32).at[ar // 8, ar].set(1.0)
  S2 = jnp.zeros((80, hid), jnp.float32).at[16 + ar, ar].set(1.0)
  W2e = jnp.concatenate(
      [W2, W2 @ a_src2.T, jnp.zeros((hid, 15), jnp.float32)], axis=1)
  W2d = jnp.concatenate(
      [W2 @ a_dst2.T, jnp.zeros((hid, 15), jnp.float32)], axis=1)
  ar16 = jnp.arange(ncls)
  S3 = jnp.zeros((32, ncls), jnp.float32).at[ar16, ar16].set(1.0)
  S4 = jnp.zeros((32, ncls), jnp.float32).at[16, :].set(1.0)
  b1row = b1.reshape(1, hid)
  b2row = b2.reshape(1, ncls)

  x_pad = jnp.pad(x, ((0, n_pad - n), (0, 0)))
  zrows80 = jnp.zeros((n_pad // 16, 80), jnp.float32)
  zrows32 = jnp.zeros((n_pad // 16, 32), jnp.float32)

  full = lambda shp: pl.BlockSpec(shp, lambda i: (0, 0))
  rowblk = lambda w: pl.BlockSpec((blk, w), lambda i: (i, 0))

  t1, t2 = pl.pallas_call(
      _prep1_body,
      grid=(n_blk,),
      in_specs=[rowblk(d), full((d, hid)), full((hid, 80)), full((hid, 16))],
      out_specs=[rowblk(80), rowblk(16)],
      out_shape=[
          jax.ShapeDtypeStruct((n_pad, 80), jnp.float32),
          jax.ShapeDtypeStruct((n_pad, 16), jnp.float32),
      ],
  )(x_pad, W1, G, G2)

  ei_s = edge_index[0]
  ei_d = edge_index[1]
  p1 = _edge1_kernel(n_pad, ept, k_chunk, t1, t2, ei_s, ei_d, zrows80)

  u1e, td2 = pl.pallas_call(
      _mid_body,
      grid=(n_blk,),
      in_specs=[rowblk(80), rowblk(80), full((80, hid)), full((80, hid)),
                pl.BlockSpec((1, hid), lambda i: (0, 0)), full((hid, 32)),
                full((hid, 16))],
      out_specs=[rowblk(32), rowblk(16)],
      out_shape=[
          jax.ShapeDtypeStruct((n_pad, 32), jnp.float32),
          jax.ShapeDtypeStruct((n_pad, 16), jnp.float32),
      ],
  )(p1[0], p1[1], S1, S2, b1row, W2e, W2d)

  p2 = _edge2_kernel(n_pad, ept, k_chunk, u1e, td2, ei_s, ei_d, zrows32)

  out = pl.pallas_call(
      _final_body,
      grid=(n_blk,),
      in_specs=[rowblk(32), rowblk(32), full((32, ncls)), full((32, ncls)),
                pl.BlockSpec((1, ncls), lambda i: (0, 0))],
      out_specs=rowblk(ncls),
      out_shape=jax.ShapeDtypeStruct((n_pad, ncls), jnp.float32),
  )(p2[0], p2[1], S3, S4, b2row)

  return out[:n]
